# Initial kernel scaffold; baseline (speedup 1.0000x reference)
import jax
import jax.numpy as jnp
from jax.experimental import pallas as pl


def kernel(x, kmer, indices, emb, W1, b1, W2, b2, W3, b3, W4, b4):
    raise NotImplementedError("write your pallas kernel here")


# speedup vs baseline: 3.3727x; 3.3727x over previous
import functools

import jax
import jax.numpy as jnp
from jax import lax
from jax.experimental import pallas as pl
from jax.experimental.pallas import tpu as pltpu
from jax.experimental.pallas import tpu_sc as plsc

N = 800000
DIM_COV = 16
EMB = 2
G = 25000
GS = 32
H = 150
VOCAB = 18


PACK = 8
NP_ROWS = N // PACK
TILE_A = 4000
KX = PACK * DIM_COV
KO = PACK * VOCAB
HO = PACK * H


def _mlp_rows_body(x_ref, km_ref, wcat_ref, r_ref, b1_ref, s_ref,
                   b2_ref, out_ref):
    c1 = jnp.dot(km_ref[:, :], r_ref[:, :],
                 preferred_element_type=jnp.float32)
    vmod = (lax.broadcasted_iota(jnp.int32, (1, KO), 1) % VOCAB
            ).astype(jnp.float32)
    oh = (c1 == vmod).astype(jnp.float32)
    z = jnp.concatenate([x_ref[:, :], oh], axis=1)
    h1 = jnp.maximum(
        jnp.dot(z, wcat_ref[:, :], preferred_element_type=jnp.float32)
        + b1_ref[:, :], 0.0)
    h2 = jnp.dot(h1, s_ref[:, :],
                 preferred_element_type=jnp.float32) + b2_ref[:, :]
    out_ref[:, :] = jnp.maximum(h2, 0.0)


def _mlp_rows(x2, kmp, wcat, r_mat, b1t, s_mat, b2r):
    grid = (NP_ROWS // TILE_A,)
    return pl.pallas_call(
        _mlp_rows_body,
        grid=grid,
        in_specs=[
            pl.BlockSpec((TILE_A, KX), lambda i: (i, 0)),
            pl.BlockSpec((TILE_A, PACK), lambda i: (i, 0)),
            pl.BlockSpec((KX + KO, HO), lambda i: (0, 0)),
            pl.BlockSpec((PACK, KO), lambda i: (0, 0)),
            pl.BlockSpec((1, HO), lambda i: (0, 0)),
            pl.BlockSpec((HO, PACK), lambda i: (0, 0)),
            pl.BlockSpec((1, 1), lambda i: (0, 0)),
        ],
        out_specs=pl.BlockSpec((TILE_A, PACK), lambda i: (i, 0)),
        out_shape=jax.ShapeDtypeStruct((NP_ROWS, PACK), jnp.float32),
    )(x2, kmp, wcat, r_mat, b1t, s_mat, b2r)


NC, NS = 2, 16
NW = NC * NS
G_PAD = 25600
GPW = G_PAD // NW
CHUNK_G = 16
N_CHUNK = GPW // CHUNK_G
IDX_PER_CHUNK = CHUNK_G * GS
STREAMS = IDX_PER_CHUNK // 128
IDX_ROWS = G_PAD * GS // 128


def _sortnet16(v):
    n = len(v)
    v = list(v)
    k = 2
    while k <= n:
        j = k // 2
        while j >= 1:
            for i in range(n):
                l = i ^ j
                if l > i:
                    a, b = v[i], v[l]
                    mn = jnp.minimum(a, b)
                    mx = jnp.maximum(a, b)
                    if (i & k) == 0:
                        v[i], v[l] = mn, mx
                    else:
                        v[i], v[l] = mx, mn
            j //= 2
        k *= 2
    return v


IDX_ALL = (N_CHUNK + 2) * IDX_PER_CHUNK
IDX_HBM = (NW - 1) * GPW * GS + IDX_ALL
STAGE_CHUNK = 25000


def _sc_compute_stats(buf, cc, st_m, st_v, st_mn, st_md, st_mx):
    lane = lax.iota(jnp.int32, 16) * GS
    vs = []
    ssum = None
    ssq = None
    smn = None
    smx = None
    for jv in range(GS):
        v = plsc.load_gather(buf, [lane + jv])
        vs.append(v)
        if jv == 0:
            ssum, ssq, smn, smx = v, v * v, v, v
        else:
            ssum = ssum + v
            ssq = ssq + v * v
            smn = jnp.minimum(smn, v)
            smx = jnp.maximum(smx, v)
    mean = ssum * (1.0 / GS)
    var = (ssq - ssum * mean) * (1.0 / (GS - 1))
    lo = _sortnet16(vs[:16])
    hi = _sortnet16(vs[16:])
    low16 = [jnp.minimum(lo[i], hi[15 - i]) for i in range(16)]
    med = low16[0]
    for i in range(1, 16):
        med = jnp.maximum(med, low16[i])
    off = cc * CHUNK_G
    st_m[pl.ds(off, 16)] = mean
    st_v[pl.ds(off, 16)] = var
    st_mn[pl.ds(off, 16)] = smn
    st_md[pl.ds(off, 16)] = med
    st_mx[pl.ds(off, 16)] = smx


def _sc_stats_body(h_ref, idx_ref, out_m, out_v, out_mn, out_md, out_mx,
                   sh_h, stage_v, idx_all, vals0, vals1,
                   st_m, st_v, st_mn, st_md, st_mx, sem0, sem1):
    wid = lax.axis_index("s") * NC + lax.axis_index("c")
    sid = lax.axis_index("s")
    g0 = wid * GPW
    stat_bufs = (st_m, st_v, st_mn, st_md, st_mx)

    nps = N // NS
    n_st = nps // STAGE_CHUNK
    for t in range(n_st):
        off = sid * nps + t * STAGE_CHUNK
        pltpu.sync_copy(h_ref.at[pl.ds(off, STAGE_CHUNK)], stage_v)
        pltpu.sync_copy(stage_v, sh_h.at[pl.ds(off, STAGE_CHUNK)])
    plsc.subcore_barrier()

    pltpu.sync_copy(idx_ref.at[pl.ds(wid * (GPW * GS), IDX_ALL)], idx_all)

    def fire(cc, buf, sem):
        for s in range(STREAMS):
            pltpu.async_copy(
                sh_h.at[idx_all.at[pl.ds(cc * IDX_PER_CHUNK + s * 128,
                                         128)]],
                buf.at[pl.ds(s * 128, 128)], sem)

    def drain(buf, sem):
        pltpu.make_async_copy(sh_h.at[pl.ds(0, IDX_PER_CHUNK)], buf,
                              sem).wait()

    fire(0, vals0, sem0)
    fire(1, vals1, sem1)

    def outer(o, carry):
        for par, (buf, sem) in enumerate(((vals0, sem0), (vals1, sem1))):
            cc = 2 * o + par
            drain(buf, sem)
            _sc_compute_stats(buf, cc, *stat_bufs)
            fire(cc + 2, buf, sem)
        return carry

    lax.fori_loop(0, N_CHUNK // 2, outer, 0)
    drain(vals0, sem0)
    drain(vals1, sem1)
    outs = (out_m, out_v, out_mn, out_md, out_mx)
    for st in range(5):
        pltpu.sync_copy(stat_bufs[st], outs[st].at[pl.ds(g0, GPW)])


def _sc_stats(h_flat, idx_flat):
    mesh = plsc.VectorSubcoreMesh(core_axis_name="c", subcore_axis_name="s",
                                  num_cores=NC, num_subcores=NS)
    f = functools.partial(
        pl.kernel,
        out_type=tuple(jax.ShapeDtypeStruct((G_PAD,), jnp.float32)
                       for _ in range(5)),
        mesh=mesh,
        scratch_types=[
            pltpu.VMEM_SHARED((N,), jnp.float32),
            pltpu.VMEM((STAGE_CHUNK,), jnp.float32),
            pltpu.VMEM((IDX_ALL,), jnp.int32),
            pltpu.VMEM((IDX_PER_CHUNK,), jnp.float32),
            pltpu.VMEM((IDX_PER_CHUNK,), jnp.float32),
            pltpu.VMEM((GPW,), jnp.float32),
            pltpu.VMEM((GPW,), jnp.float32),
            pltpu.VMEM((GPW,), jnp.float32),
            pltpu.VMEM((GPW,), jnp.float32),
            pltpu.VMEM((GPW,), jnp.float32),
            pltpu.SemaphoreType.DMA,
            pltpu.SemaphoreType.DMA,
        ],
        compiler_params=pltpu.CompilerParams(needs_layout_passes=False),
    )(_sc_stats_body)
    return f(h_flat, idx_flat)


TILE_C = 1024


def _group_mlp_body(sm_ref, sv_ref, smn_ref, smd_ref, smx_ref,
                    w3_ref, b3_ref, w4_ref, b4_ref, out_ref):
    st = jnp.concatenate(
        [sm_ref[:, :], sv_ref[:, :], smn_ref[:, :], smd_ref[:, :],
         smx_ref[:, :]], axis=0)
    a = jnp.dot(w3_ref[:, :], st,
                preferred_element_type=jnp.float32) + b3_ref[:, :]
    a = jnp.maximum(a, 0.0)
    o = jnp.dot(w4_ref[:, :], a,
                preferred_element_type=jnp.float32) + b4_ref[:, :]
    out_ref[:, :] = jax.nn.sigmoid(o)


def _group_mlp(stats5, w3, b3r, w4, b4r):
    grid = (G_PAD // TILE_C,)
    stat_spec = pl.BlockSpec((1, TILE_C), lambda i: (0, i))
    return pl.pallas_call(
        _group_mlp_body,
        grid=grid,
        in_specs=[
            stat_spec, stat_spec, stat_spec, stat_spec, stat_spec,
            pl.BlockSpec((H, 5), lambda i: (0, 0)),
            pl.BlockSpec((H, 1), lambda i: (0, 0)),
            pl.BlockSpec((1, H), lambda i: (0, 0)),
            pl.BlockSpec((1, 1), lambda i: (0, 0)),
        ],
        out_specs=pl.BlockSpec((1, TILE_C), lambda i: (0, i)),
        out_shape=jax.ShapeDtypeStruct((1, G_PAD), jnp.float32),
    )(*stats5, w3, b3r, w4, b4r)


def kernel(x, kmer, indices, emb, W1, b1, W2, b2, W3, b3, W4, b4):
    eye = jnp.eye(PACK, dtype=jnp.float32)
    x2 = x.reshape(NP_ROWS, KX)
    kmp = kmer.astype(jnp.float32).reshape(NP_ROWS, PACK)
    a1 = W1[:, :DIM_COV].T
    e1 = emb @ W1[:, DIM_COV:].T
    wbd = jnp.kron(eye, a1)
    ebd = jnp.kron(eye, e1)
    wcat = jnp.concatenate([wbd, ebd], axis=0)
    r_mat = jnp.kron(eye, jnp.ones((1, VOCAB), jnp.float32))
    b1t = jnp.tile(b1, PACK).reshape(1, HO)
    s_mat = jnp.kron(eye, W2.T)
    b2r = b2.reshape(1, 1)

    h = _mlp_rows(x2, kmp, wcat, r_mat, b1t, s_mat, b2r)
    h_flat = h.reshape(N)

    idx = indices.astype(jnp.int32)
    idx = jnp.pad(idx, ((0, G_PAD - G), (0, 0)))
    idx_flat = jnp.pad(idx.reshape(G_PAD * GS),
                       (0, IDX_HBM - G_PAD * GS))

    stats5 = _sc_stats(h_flat, idx_flat)
    stats5 = tuple(s.reshape(1, G_PAD) for s in stats5)

    b3r = b3.reshape(H, 1)
    b4r = b4.reshape(1, 1)
    out = _group_mlp(stats5, W3, b3r, W4, b4r)
    return out[0, :G].reshape(G, 1)
